# Initial kernel scaffold; baseline (speedup 1.0000x reference)
import jax
import jax.numpy as jnp
from jax.experimental import pallas as pl


def kernel(x, pos, batch, x_skip, pos_skip, batch_skip, W1, b1, W2, b2):
    raise NotImplementedError("write your pallas kernel here")


# speedup vs baseline: 10.1645x; 10.1645x over previous
import functools

import jax
import jax.numpy as jnp
from jax import lax
from jax.experimental import pallas as pl
from jax.experimental.pallas import tpu as pltpu
from jax.experimental.pallas import tpu_sc as plsc

_BIG = 1e10
_K = 3


def _knn_body(pos_t_ref, ps_ref, batch_ref, bs_ref, idx_ref, wn_ref):
    t = ps_ref.shape[0]
    n = pos_t_ref.shape[1]
    q = ps_ref[...]
    kx = pos_t_ref[0:1, :]
    ky = pos_t_ref[1:2, :]
    kz = pos_t_ref[2:3, :]
    dx = q[:, 0:1] - kx
    dy = q[:, 1:2] - ky
    dz = q[:, 2:3] - kz
    d2 = (dx * dx + dy * dy) + dz * dz
    mask = bs_ref[...] != batch_ref[...]
    d2m = jnp.where(mask, _BIG, d2)
    colidx = lax.broadcasted_iota(jnp.int32, (t, n), 1)
    sentinel = 2**30
    idxs = []
    d2s = []
    for _ in range(_K):
        m = jnp.min(d2m, axis=1, keepdims=True)
        eq = d2m == m
        sel = jnp.min(jnp.where(eq, colidx, sentinel), axis=1, keepdims=True)
        idxs.append(sel)
        d2s.append(m)
        d2m = jnp.where(colidx == sel, _BIG, d2m)
    w = [1.0 / jnp.maximum(d, 1e-16) for d in d2s]
    den = (w[0] + w[1]) + w[2]
    idx_ref[...] = jnp.concatenate(idxs, axis=1)
    wn_ref[...] = jnp.concatenate([wj / den for wj in w], axis=1)


def _knn_topk(pos_t, pos_skip, batch_row, bs_col, tile=256):
    m = pos_skip.shape[0]
    n = pos_t.shape[1]
    grid = (m // tile,)
    return pl.pallas_call(
        _knn_body,
        grid=grid,
        in_specs=[
            pl.BlockSpec((3, n), lambda i: (0, 0)),
            pl.BlockSpec((tile, 3), lambda i: (i, 0)),
            pl.BlockSpec((1, n), lambda i: (0, 0)),
            pl.BlockSpec((tile, 1), lambda i: (i, 0)),
        ],
        out_specs=[
            pl.BlockSpec((tile, _K), lambda i: (i, 0)),
            pl.BlockSpec((tile, _K), lambda i: (i, 0)),
        ],
        out_shape=[
            jax.ShapeDtypeStruct((m, _K), jnp.int32),
            jax.ShapeDtypeStruct((m, _K), jnp.float32),
        ],
    )(pos_t, pos_skip, batch_row, bs_col)


_NC = 2
_NS = 16
_NW = _NC * _NS


def _sc_gather(x, idx_flat):
    mk = idx_flat.shape[0]
    d = x.shape[1]
    per_w = mk // _NW
    chunk = 128
    n_chunks = per_w // chunk
    mesh = plsc.VectorSubcoreMesh(core_axis_name="c", subcore_axis_name="s")

    @functools.partial(
        pl.kernel,
        mesh=mesh,
        out_type=jax.ShapeDtypeStruct((mk, d), jnp.float32),
        scratch_types=[
            pltpu.VMEM((chunk,), jnp.int32),
            pltpu.VMEM((chunk, d), jnp.float32),
            pltpu.SemaphoreType.DMA,
        ],
    )
    def gather_kernel(x_hbm, idx_hbm, out_hbm, idx_v, rows_v, sem):
        wid = lax.axis_index("s") * _NC + lax.axis_index("c")
        for c in range(n_chunks):
            base = wid * per_w + c * chunk
            pltpu.sync_copy(idx_hbm.at[pl.ds(base, chunk)], idx_v)
            pltpu.async_copy(x_hbm.at[idx_v], rows_v, sem).wait()
            pltpu.sync_copy(rows_v, out_hbm.at[pl.ds(base, chunk)])

    return gather_kernel(x, idx_flat)


def _mlp_body(xg0_ref, xg1_ref, xg2_ref, wn_ref, xs_ref, w1a_ref, w1b_ref,
              b1_ref, w2_ref, b2_ref, out_ref):
    interp = (wn_ref[:, 0:1] * xg0_ref[...]
              + wn_ref[:, 1:2] * xg1_ref[...]
              + wn_ref[:, 2:3] * xg2_ref[...])
    h = jnp.dot(interp, w1a_ref[...], preferred_element_type=jnp.float32,
                precision=lax.Precision.HIGHEST)
    h += jnp.dot(xs_ref[...], w1b_ref[...], preferred_element_type=jnp.float32,
                 precision=lax.Precision.HIGHEST)
    h = jnp.maximum(h + b1_ref[...], 0.0)
    out = jnp.dot(h, w2_ref[...], preferred_element_type=jnp.float32,
                  precision=lax.Precision.HIGHEST)
    out_ref[...] = out + b2_ref[...]


def _interp_mlp(xg_flat, wn, x_skip, w1a, w1b, b1, w2, b2, tile=512):
    d_in = xg_flat.shape[1]
    m = xg_flat.shape[0] // _K
    d_skip = x_skip.shape[1]
    d_hid = w1a.shape[1]
    d_out = w2.shape[1]
    grid = (m // tile,)
    nt = m // tile
    return pl.pallas_call(
        _mlp_body,
        grid=grid,
        in_specs=[
            pl.BlockSpec((tile, d_in), lambda i: (i, 0)),
            pl.BlockSpec((tile, d_in), lambda i: (i + nt, 0)),
            pl.BlockSpec((tile, d_in), lambda i: (i + 2 * nt, 0)),
            pl.BlockSpec((tile, _K), lambda i: (i, 0)),
            pl.BlockSpec((tile, d_skip), lambda i: (i, 0)),
            pl.BlockSpec((d_in, d_hid), lambda i: (0, 0)),
            pl.BlockSpec((d_skip, d_hid), lambda i: (0, 0)),
            pl.BlockSpec((1, d_hid), lambda i: (0, 0)),
            pl.BlockSpec((d_hid, d_out), lambda i: (0, 0)),
            pl.BlockSpec((1, d_out), lambda i: (0, 0)),
        ],
        out_specs=pl.BlockSpec((tile, d_out), lambda i: (i, 0)),
        out_shape=jax.ShapeDtypeStruct((m, d_out), jnp.float32),
    )(xg_flat, xg_flat, xg_flat, wn, x_skip, w1a, w1b, b1, w2, b2)


def kernel(x, pos, batch, x_skip, pos_skip, batch_skip, W1, b1, W2, b2):
    n, d_in = x.shape
    m = pos_skip.shape[0]
    pos_t = pos.T
    batch_row = batch.astype(jnp.int32).reshape(1, n)
    bs_col = batch_skip.astype(jnp.int32).reshape(m, 1)
    idx, wn = _knn_topk(pos_t, pos_skip, batch_row, bs_col)
    xg_flat = _sc_gather(x, idx.T.reshape(-1))
    out = _interp_mlp(xg_flat, wn, x_skip, W1[:d_in], W1[d_in:],
                      b1.reshape(1, -1), W2, b2.reshape(1, -1))
    return (out, pos_skip, batch_skip)
